# Initial kernel scaffold; baseline (speedup 1.0000x reference)
import jax
import jax.numpy as jnp
from jax.experimental import pallas as pl


def kernel(rays_o, rays_d, rays_id, rvec, tvec):
    raise NotImplementedError("write your pallas kernel here")


# speedup vs baseline: 3.7754x; 3.7754x over previous
import jax
import jax.numpy as jnp
from jax import lax
from jax.experimental import pallas as pl
from jax.experimental.pallas import tpu as pltpu
from jax.experimental.pallas import tpu_sc as plsc

N_RAYS = 1048576
NUM_CAMS = 1000
CAM_PAD = 1024
NC = 2
NS = 16
L = 16
NW = NC * NS
RAYS_PER_W = N_RAYS // NW
CHUNK = 2048
NCHUNKS = RAYS_PER_W // CHUNK
GROUPS = CHUNK // L


def _full(v):
    return jnp.full((L,), v, dtype=jnp.int32)


def _body(rays_o_hbm, rays_d_hbm, ids_hbm, rvec_hbm, tvec_hbm,
          out_o_hbm, out_d_hbm,
          rvec_v, tvec_v, tbl_v, o_v, d_v, ids_v, oo_v, dd_v):
    wid = lax.axis_index("s") * NC + lax.axis_index("c")
    iota = lax.iota(jnp.int32, L)

    pltpu.sync_copy(rvec_hbm, rvec_v.at[pl.ds(0, NUM_CAMS * 4)])
    pltpu.sync_copy(tvec_hbm, tvec_v.at[pl.ds(0, NUM_CAMS * 3)])

    def build(g, carry):
        base = g * L
        cam4 = (base + iota) * 4
        cam3 = (base + iota) * 3
        x = plsc.load_gather(rvec_v, [cam4])
        y = plsc.load_gather(rvec_v, [cam4 + _full(1)])
        z = plsc.load_gather(rvec_v, [cam4 + _full(2)])
        w = plsc.load_gather(rvec_v, [cam4 + _full(3)])
        t0 = plsc.load_gather(tvec_v, [cam3])
        t1 = plsc.load_gather(tvec_v, [cam3 + _full(1)])
        t2 = plsc.load_gather(tvec_v, [cam3 + _full(2)])
        theta2 = 1e-5 + x * x + y * y + z * z + w * w
        a = 2.0 / theta2
        axx = a * x * x
        ayy = a * y * y
        azz = a * z * z
        axy = a * x * y
        axz = a * x * z
        ayz = a * y * z
        axw = a * x * w
        ayw = a * y * w
        azw = a * z * w
        tbl_v[pl.ds(0 * CAM_PAD + base, L)] = 1.0 - ayy - azz
        tbl_v[pl.ds(1 * CAM_PAD + base, L)] = axy - azw
        tbl_v[pl.ds(2 * CAM_PAD + base, L)] = axz + ayw
        tbl_v[pl.ds(3 * CAM_PAD + base, L)] = axy + azw
        tbl_v[pl.ds(4 * CAM_PAD + base, L)] = 1.0 - axx - azz
        tbl_v[pl.ds(5 * CAM_PAD + base, L)] = ayz - axw
        tbl_v[pl.ds(6 * CAM_PAD + base, L)] = axz - ayw
        tbl_v[pl.ds(7 * CAM_PAD + base, L)] = ayz + axw
        tbl_v[pl.ds(8 * CAM_PAD + base, L)] = 1.0 - axx - ayy
        tbl_v[pl.ds(9 * CAM_PAD + base, L)] = t0
        tbl_v[pl.ds(10 * CAM_PAD + base, L)] = t1
        tbl_v[pl.ds(11 * CAM_PAD + base, L)] = t2
        return carry

    lax.fori_loop(0, CAM_PAD // L, build, 0)

    wbase = wid * RAYS_PER_W

    def chunk_body(c, carry):
        base = wbase + c * CHUNK
        pltpu.sync_copy(rays_o_hbm.at[pl.ds(base * 4, CHUNK * 4)], o_v)
        pltpu.sync_copy(rays_d_hbm.at[pl.ds(base * 4, CHUNK * 4)], d_v)
        pltpu.sync_copy(ids_hbm.at[pl.ds(base, CHUNK)], ids_v)

        def group(g, carry2):
            r = g * L
            r4 = r * 4 + iota * 4
            r3 = r * 3 + iota * 3
            ids16 = ids_v[pl.ds(r, L)]
            o0 = plsc.load_gather(o_v, [r4])
            o1 = plsc.load_gather(o_v, [r4 + _full(1)])
            o2 = plsc.load_gather(o_v, [r4 + _full(2)])
            d0 = plsc.load_gather(d_v, [r4])
            d1 = plsc.load_gather(d_v, [r4 + _full(1)])
            d2 = plsc.load_gather(d_v, [r4 + _full(2)])
            c00 = plsc.load_gather(tbl_v, [ids16])
            c01 = plsc.load_gather(tbl_v, [ids16 + _full(1 * CAM_PAD)])
            c02 = plsc.load_gather(tbl_v, [ids16 + _full(2 * CAM_PAD)])
            c10 = plsc.load_gather(tbl_v, [ids16 + _full(3 * CAM_PAD)])
            c11 = plsc.load_gather(tbl_v, [ids16 + _full(4 * CAM_PAD)])
            c12 = plsc.load_gather(tbl_v, [ids16 + _full(5 * CAM_PAD)])
            c20 = plsc.load_gather(tbl_v, [ids16 + _full(6 * CAM_PAD)])
            c21 = plsc.load_gather(tbl_v, [ids16 + _full(7 * CAM_PAD)])
            c22 = plsc.load_gather(tbl_v, [ids16 + _full(8 * CAM_PAD)])
            t0 = plsc.load_gather(tbl_v, [ids16 + _full(9 * CAM_PAD)])
            t1 = plsc.load_gather(tbl_v, [ids16 + _full(10 * CAM_PAD)])
            t2 = plsc.load_gather(tbl_v, [ids16 + _full(11 * CAM_PAD)])
            plsc.store_scatter(oo_v, [r3], o0 + t0)
            plsc.store_scatter(oo_v, [r3 + _full(1)], o1 + t1)
            plsc.store_scatter(oo_v, [r3 + _full(2)], o2 + t2)
            plsc.store_scatter(dd_v, [r3], d0 * c00 + d1 * c01 + d2 * c02)
            plsc.store_scatter(dd_v, [r3 + _full(1)],
                               d0 * c10 + d1 * c11 + d2 * c12)
            plsc.store_scatter(dd_v, [r3 + _full(2)],
                               d0 * c20 + d1 * c21 + d2 * c22)
            return carry2

        lax.fori_loop(0, GROUPS, group, 0)
        pltpu.sync_copy(oo_v, out_o_hbm.at[pl.ds(base * 3, CHUNK * 3)])
        pltpu.sync_copy(dd_v, out_d_hbm.at[pl.ds(base * 3, CHUNK * 3)])
        return carry

    lax.fori_loop(0, NCHUNKS, chunk_body, 0)


_sc_kernel = pl.kernel(
    _body,
    out_type=(jax.ShapeDtypeStruct((N_RAYS * 3,), jnp.float32),
              jax.ShapeDtypeStruct((N_RAYS * 3,), jnp.float32)),
    mesh=plsc.VectorSubcoreMesh(core_axis_name="c", subcore_axis_name="s"),
    compiler_params=pltpu.CompilerParams(needs_layout_passes=False),
    scratch_types=[
        pltpu.VMEM((CAM_PAD * 4,), jnp.float32),
        pltpu.VMEM((CAM_PAD * 3,), jnp.float32),
        pltpu.VMEM((12 * CAM_PAD,), jnp.float32),
        pltpu.VMEM((CHUNK * 4,), jnp.float32),
        pltpu.VMEM((CHUNK * 4,), jnp.float32),
        pltpu.VMEM((CHUNK,), jnp.int32),
        pltpu.VMEM((CHUNK * 3,), jnp.float32),
        pltpu.VMEM((CHUNK * 3,), jnp.float32),
    ],
)


def kernel(rays_o, rays_d, rays_id, rvec, tvec):
    ids = rays_id.astype(jnp.int32)
    out_o, out_d = _sc_kernel(rays_o.reshape(-1), rays_d.reshape(-1), ids,
                              rvec.reshape(-1), tvec.reshape(-1))
    return (out_o.reshape(N_RAYS, 3), out_d.reshape(N_RAYS, 3))
